# baseline (speedup 1.0000x reference)
import functools

import jax
import jax.numpy as jnp
from jax import lax
from jax.experimental import pallas as pl
from jax.experimental.pallas import tpu as pltpu
from jax.experimental.pallas import tpu_sc as plsc

_B, _L, _D = 4096, 200, 64
_N = _B * _L
_NC, _NS = 2, 16
_NW = _NC * _NS
_PER_W = _N // _NW
_CHUNK = 128
_NCH = _PER_W // _CHUNK

_mesh = plsc.VectorSubcoreMesh(core_axis_name="c", subcore_axis_name="s")


@functools.partial(
    pl.kernel,
    mesh=_mesh,
    out_type=jax.ShapeDtypeStruct((_N, _D), jnp.float32),
    scratch_types=[
        pltpu.VMEM((_NCH, _CHUNK), jnp.int32),
        pltpu.VMEM((2, _CHUNK, _D), jnp.float32),
        pltpu.SemaphoreType.DMA,
        pltpu.SemaphoreType.DMA,
    ],
    compiler_params=pltpu.CompilerParams(use_tc_tiling_on_sc=False),
)
def _emb(idx_hbm, tab_hbm, out_hbm, idx_v, rows_v, gsem, ssem):
    wid = lax.axis_index("s") * _NC + lax.axis_index("c")
    base = wid * _PER_W
    pltpu.sync_copy(idx_hbm.at[wid], idx_v)

    def chunk(j, carry):
        pltpu.async_copy(tab_hbm.at[idx_v.at[j]], rows_v.at[0], gsem).wait()
        pltpu.sync_copy(rows_v.at[0],
                        out_hbm.at[pl.ds(base + j * _CHUNK, _CHUNK)])
        return carry

    lax.fori_loop(0, _NCH, chunk, 0)


def kernel(word_indices, table):
    idx = word_indices.reshape(_NW, _NCH, _CHUNK).astype(jnp.int32)
    out = _emb(idx, table)
    return out.reshape(_B, _L, _D)


# speedup vs baseline: 1.1128x; 1.1128x over previous
import functools

import jax
import jax.numpy as jnp
from jax import lax
from jax.experimental import pallas as pl
from jax.experimental.pallas import tpu as pltpu
from jax.experimental.pallas import tpu_sc as plsc

_B, _L, _D = 4096, 200, 64
_N = _B * _L
_NC, _NS = 2, 16
_NW = _NC * _NS
_PER_W = _N // _NW
_CHUNK = 128
_NCH = _PER_W // _CHUNK
_K = 4
_NG = _NCH // _K

_mesh = plsc.VectorSubcoreMesh(core_axis_name="c", subcore_axis_name="s")


@functools.partial(
    pl.kernel,
    mesh=_mesh,
    out_type=jax.ShapeDtypeStruct((_N, _D), jnp.float32),
    scratch_types=[
        pltpu.VMEM((_NCH, _CHUNK), jnp.int32),
        pltpu.VMEM((2 * _K, _CHUNK, _D), jnp.float32),
        pltpu.SemaphoreType.DMA,
        pltpu.SemaphoreType.DMA,
        pltpu.SemaphoreType.DMA,
    ],
    compiler_params=pltpu.CompilerParams(use_tc_tiling_on_sc=False),
)
def _emb(idx_hbm, tab_hbm, out_hbm, idx_v, rows_v, gsem, ssem0, ssem1):
    wid = lax.axis_index("s") * _NC + lax.axis_index("c")
    base = wid * _PER_W
    pltpu.sync_copy(idx_hbm.at[wid], idx_v)

    ssems = (ssem0, ssem1)

    def fire_g(g0, half):
        for b in range(_K):
            pltpu.async_copy(tab_hbm.at[idx_v.at[g0 + b]],
                             rows_v.at[half * _K + b], gsem)

    def drain_g(half):
        for b in range(_K):
            pltpu.make_async_copy(tab_hbm.at[idx_v.at[0]],
                                  rows_v.at[half * _K + b], gsem).wait()

    def fire_s(g0, half):
        for b in range(_K):
            pltpu.async_copy(
                rows_v.at[half * _K + b],
                out_hbm.at[pl.ds(base + (g0 + b) * _CHUNK, _CHUNK)],
                ssems[half])

    def drain_s(half):
        for b in range(_K):
            pltpu.make_async_copy(rows_v.at[half * _K + b],
                                  out_hbm.at[pl.ds(base, _CHUNK)],
                                  ssems[half]).wait()

    fire_g(0, 0)
    drain_g(0)
    fire_s(0, 0)
    fire_g(_K, 1)

    @pl.loop(1, _NG - 1, step=2)
    def _(gi):
        for p in range(2):
            g = gi + p
            half = (1 + p) % 2
            drain_g(half)
            fire_s(g * _K, half)
            drain_s(1 - half)
            fire_g((g + 1) * _K, 1 - half)

    drain_g(1)
    fire_s((_NG - 1) * _K, 1)
    drain_s(0)
    drain_s(1)


def kernel(word_indices, table):
    idx = word_indices.reshape(_NW, _NCH, _CHUNK).astype(jnp.int32)
    out = _emb(idx, table)
    return out.reshape(_B, _L, _D)
